# baseline (speedup 1.0000x reference)
import functools

import jax
import jax.numpy as jnp
from jax import lax
from jax.experimental import pallas as pl
from jax.experimental.pallas import tpu as pltpu
from jax.experimental.pallas import tpu_sc as plsc

_D = 64
_LANES = 16


def _sc_bow_body(idx_hbm, emb_hbm, freq_hbm, out_hbm,
                 idx_v, rows_v, f_v, acc_v, sem, *, n_chunks):
    cid = lax.axis_index("c")
    sid = lax.axis_index("s")
    wid = sid * 2 + cid

    @pl.when(wid < n_chunks)
    def _():
        off = pl.multiple_of(wid * _LANES, 8)
        pltpu.sync_copy(idx_hbm.at[pl.ds(off, _LANES)], idx_v)
        fcp = pltpu.async_copy(freq_hbm.at[idx_v], f_v, sem)
        ecp = pltpu.async_copy(emb_hbm.at[idx_v], rows_v, sem)
        fcp.wait()
        ecp.wait()

        wvec = 1.0 / f_v[...]
        accs = tuple(jnp.zeros((_LANES,), jnp.float32)
                     for _ in range(_D // _LANES))
        for j in range(_LANES):
            w = wvec[j]
            accs = tuple(
                accs[c] + w * rows_v[j, pl.ds(c * _LANES, _LANES)]
                for c in range(_D // _LANES)
            )
        for c in range(_D // _LANES):
            acc_v[pl.ds(c * _LANES, _LANES)] = accs[c]
        out_off = pl.multiple_of(wid * _D, 8)
        pltpu.sync_copy(acc_v, out_hbm.at[pl.ds(out_off, _D)])


def _make_sc_bow(n_chunks):
    return functools.partial(
        pl.kernel,
        out_type=jax.ShapeDtypeStruct((n_chunks * _D,), jnp.float32),
        mesh=plsc.VectorSubcoreMesh(core_axis_name="c", subcore_axis_name="s"),
        scratch_types=[
            pltpu.VMEM((_LANES,), jnp.int32),
            pltpu.VMEM((_LANES, _D), jnp.float32),
            pltpu.VMEM((_LANES,), jnp.float32),
            pltpu.VMEM((_D,), jnp.float32),
            pltpu.SemaphoreType.DMA,
        ],
        compiler_params=pltpu.CompilerParams(use_tc_tiling_on_sc=False),
    )(functools.partial(_sc_bow_body, n_chunks=n_chunks))


def _tc_head_body(parts_ref, w_ref, b_ref, out_ref, *, scale):
    bow = jnp.sum(parts_ref[...], axis=0, keepdims=True) * scale
    logits = lax.dot_general(
        bow, w_ref[...], (((1,), (1,)), ((), ())),
        preferred_element_type=jnp.float32) + b_ref[...]
    m = jnp.max(logits, axis=-1, keepdims=True)
    s = logits - m
    lse = jnp.log(jnp.sum(jnp.exp(s), axis=-1, keepdims=True))
    out_ref[...] = s - lse


def kernel(input, emb_tensor, freq, W, b):
    L = input.shape[1]
    l_pad = ((L + _LANES - 1) // _LANES) * _LANES
    n_chunks = l_pad // _LANES
    idx = jnp.concatenate(
        [input[0], jnp.zeros((l_pad - L,), jnp.int32)])
    parts = _make_sc_bow(n_chunks)(idx, emb_tensor, freq)

    scale = 1.0 / (float(L) * 100000.0)
    out = pl.pallas_call(
        functools.partial(_tc_head_body, scale=scale),
        out_shape=jax.ShapeDtypeStruct((1, 2), jnp.float32),
    )(parts.reshape(n_chunks, _D), W, b.reshape(1, 2))
    return out


# speedup vs baseline: 1.0012x; 1.0012x over previous
import functools

import jax
import jax.numpy as jnp
from jax import lax
from jax.experimental import pallas as pl
from jax.experimental.pallas import tpu as pltpu
from jax.experimental.pallas import tpu_sc as plsc

_D = 64
_LANES = 16


def _sc_bow_body(idx_hbm, emb2_hbm, freq_hbm, out_hbm,
                 idx_v, pidx_v, rows_v, f_v, acc_v, sem, *, n_chunks):
    cid = lax.axis_index("c")
    sid = lax.axis_index("s")
    wid = sid * 2 + cid

    @pl.when(wid < n_chunks)
    def _():
        off = pl.multiple_of(wid * _LANES, 8)
        pltpu.sync_copy(idx_hbm.at[pl.ds(off, _LANES)], idx_v)
        fcp = pltpu.async_copy(freq_hbm.at[idx_v], f_v, sem)
        ivec = idx_v[...]
        pidx_v[...] = jax.lax.shift_right_logical(ivec, 1)
        ecp = pltpu.async_copy(emb2_hbm.at[pidx_v], rows_v, sem)
        hvec = jax.lax.bitwise_and(ivec, 1) * _D
        fcp.wait()
        ecp.wait()

        wvec = 1.0 / f_v[...]
        accs = tuple(jnp.zeros((_LANES,), jnp.float32)
                     for _ in range(_D // _LANES))
        for j in range(_LANES):
            w = wvec[j]
            h = hvec[j]
            accs = tuple(
                accs[c] + w * rows_v[j, pl.ds(h + c * _LANES, _LANES)]
                for c in range(_D // _LANES)
            )
        for c in range(_D // _LANES):
            acc_v[pl.ds(c * _LANES, _LANES)] = accs[c]
        out_off = pl.multiple_of(wid * _D, 8)
        pltpu.sync_copy(acc_v, out_hbm.at[pl.ds(out_off, _D)])


def _make_sc_bow(n_chunks):
    return functools.partial(
        pl.kernel,
        out_type=jax.ShapeDtypeStruct((n_chunks * _D,), jnp.float32),
        mesh=plsc.VectorSubcoreMesh(core_axis_name="c", subcore_axis_name="s"),
        scratch_types=[
            pltpu.VMEM((_LANES,), jnp.int32),
            pltpu.VMEM((_LANES,), jnp.int32),
            pltpu.VMEM((_LANES, 2 * _D), jnp.float32),
            pltpu.VMEM((_LANES,), jnp.float32),
            pltpu.VMEM((_D,), jnp.float32),
            pltpu.SemaphoreType.DMA,
        ],
        compiler_params=pltpu.CompilerParams(use_tc_tiling_on_sc=True),
    )(functools.partial(_sc_bow_body, n_chunks=n_chunks))


def _tc_head_body(parts_ref, w_ref, b_ref, out_ref, *, scale):
    bow = jnp.sum(parts_ref[...], axis=0, keepdims=True) * scale
    logits = lax.dot_general(
        bow, w_ref[...], (((1,), (1,)), ((), ())),
        preferred_element_type=jnp.float32) + b_ref[...]
    m = jnp.max(logits, axis=-1, keepdims=True)
    s = logits - m
    lse = jnp.log(jnp.sum(jnp.exp(s), axis=-1, keepdims=True))
    out_ref[...] = s - lse


def kernel(input, emb_tensor, freq, W, b):
    L = input.shape[1]
    l_pad = ((L + _LANES - 1) // _LANES) * _LANES
    n_chunks = l_pad // _LANES
    idx = jnp.concatenate(
        [input[0], jnp.zeros((l_pad - L,), jnp.int32)])
    emb2 = emb_tensor.reshape(emb_tensor.shape[0] // 2, 2 * _D)
    parts = _make_sc_bow(n_chunks)(idx, emb2, freq)

    scale = 1.0 / (float(L) * 100000.0)
    out = pl.pallas_call(
        functools.partial(_tc_head_body, scale=scale),
        out_shape=jax.ShapeDtypeStruct((1, 2), jnp.float32),
    )(parts.reshape(n_chunks, _D), W, b.reshape(1, 2))
    return out


# speedup vs baseline: 2.4552x; 2.4522x over previous
import functools

import jax
import jax.numpy as jnp
from jax import lax
from jax.experimental import pallas as pl
from jax.experimental.pallas import tpu as pltpu
from jax.experimental.pallas import tpu_sc as plsc

_D = 64
_LANES = 16


def _sc_bow_body(idx_hbm, emb3_hbm, freq_hbm, out_hbm,
                 idx_v, tiles_v, f_v, acc_v, sem, *, l_pad):
    cid = lax.axis_index("c")
    sid = lax.axis_index("s")

    @pl.when(jnp.logical_and(cid == 0, sid == 0))
    def _():
        pltpu.sync_copy(idx_hbm, idx_v)
        frq_cp = pltpu.async_copy(freq_hbm.at[idx_v], f_v, sem)

        frq_cp.wait()

        def body(k, accs):
            base = k * _LANES
            ivec = idx_v[pl.ds(base, _LANES)]
            tvec = jax.lax.shift_right_logical(ivec, 3)
            rvec = jax.lax.bitwise_and(ivec, 7)
            wvec = 1.0 / f_v[pl.ds(base, _LANES)]
            cps = []
            for j in range(_LANES):
                cps.append(pltpu.async_copy(
                    emb3_hbm.at[tvec[j]], tiles_v.at[j], sem))
            for cp in cps:
                cp.wait()
            for j in range(_LANES):
                w = wvec[j]
                r = rvec[j]
                accs = tuple(
                    accs[c] + w * tiles_v[j, r, pl.ds(c * _LANES, _LANES)]
                    for c in range(_D // _LANES)
                )
            return accs

        init = tuple(jnp.zeros((_LANES,), jnp.float32)
                     for _ in range(_D // _LANES))
        accs = lax.fori_loop(0, l_pad // _LANES, body, init)
        for c in range(_D // _LANES):
            acc_v[pl.ds(c * _LANES, _LANES)] = accs[c]
        pltpu.sync_copy(acc_v, out_hbm)


def _make_sc_bow(l_pad):
    return functools.partial(
        pl.kernel,
        out_type=jax.ShapeDtypeStruct((_D,), jnp.float32),
        mesh=plsc.VectorSubcoreMesh(core_axis_name="c", subcore_axis_name="s"),
        scratch_types=[
            pltpu.VMEM((l_pad,), jnp.int32),
            pltpu.VMEM((_LANES, 8, _D), jnp.float32),
            pltpu.VMEM((l_pad,), jnp.float32),
            pltpu.VMEM((_D,), jnp.float32),
            pltpu.SemaphoreType.DMA,
        ],
        compiler_params=pltpu.CompilerParams(use_tc_tiling_on_sc=True),
    )(functools.partial(_sc_bow_body, l_pad=l_pad))


def _tc_head_body(bow_ref, w_ref, b_ref, out_ref, *, scale):
    bow = bow_ref[...] * scale
    logits = lax.dot_general(
        bow, w_ref[...], (((1,), (1,)), ((), ())),
        preferred_element_type=jnp.float32) + b_ref[...]
    m = jnp.max(logits, axis=-1, keepdims=True)
    s = logits - m
    lse = jnp.log(jnp.sum(jnp.exp(s), axis=-1, keepdims=True))
    out_ref[...] = s - lse


def kernel(input, emb_tensor, freq, W, b):
    L = input.shape[1]
    V = emb_tensor.shape[0]
    l_pad = ((L + _LANES - 1) // _LANES) * _LANES
    idx = jnp.concatenate(
        [input[0], jnp.zeros((l_pad - L,), jnp.int32)])
    emb3 = emb_tensor.reshape(V // 8, 8, _D)
    bow = _make_sc_bow(l_pad)(idx, emb3, freq)

    scale = 1.0 / (float(L) * 100000.0)
    out = pl.pallas_call(
        functools.partial(_tc_head_body, scale=scale),
        out_shape=jax.ShapeDtypeStruct((1, 2), jnp.float32),
    )(bow.reshape(1, _D), W, b.reshape(1, 2))
    return out


# speedup vs baseline: 2.5988x; 1.0585x over previous
import functools

import jax
import jax.numpy as jnp
from jax import lax
from jax.experimental import pallas as pl
from jax.experimental.pallas import tpu as pltpu
from jax.experimental.pallas import tpu_sc as plsc

_D = 64
_LANES = 16


def _sc_bow_body(idx_hbm, emb3_hbm, freq_hbm, out_hbm,
                 idx_v, tiles_v, f_v, acc_v, sem, *, n_chunks):
    cid = lax.axis_index("c")
    sid = lax.axis_index("s")
    wid = sid * 2 + cid

    @pl.when(wid < n_chunks)
    def _():
        off = pl.multiple_of(wid * _LANES, 8)
        pltpu.sync_copy(idx_hbm.at[pl.ds(off, _LANES)], idx_v)
        frq_cp = pltpu.async_copy(freq_hbm.at[idx_v], f_v, sem)
        ivec = idx_v[...]
        tvec = jax.lax.shift_right_logical(ivec, 3)
        rvec = jax.lax.bitwise_and(ivec, 7)
        cps = []
        for j in range(_LANES):
            cps.append(pltpu.async_copy(
                emb3_hbm.at[tvec[j]], tiles_v.at[j], sem))
        frq_cp.wait()
        for cp in cps:
            cp.wait()

        wvec = 1.0 / f_v[...]
        accs = tuple(jnp.zeros((_LANES,), jnp.float32)
                     for _ in range(_D // _LANES))
        for j in range(_LANES):
            w = wvec[j]
            r = rvec[j]
            accs = tuple(
                accs[c] + w * tiles_v[j, r, pl.ds(c * _LANES, _LANES)]
                for c in range(_D // _LANES)
            )
        for c in range(_D // _LANES):
            acc_v[pl.ds(c * _LANES, _LANES)] = accs[c]
        out_off = pl.multiple_of(wid * _D, 8)
        pltpu.sync_copy(acc_v, out_hbm.at[pl.ds(out_off, _D)])


def _make_sc_bow(n_chunks):
    return functools.partial(
        pl.kernel,
        out_type=jax.ShapeDtypeStruct((n_chunks * _D,), jnp.float32),
        mesh=plsc.VectorSubcoreMesh(core_axis_name="c", subcore_axis_name="s"),
        scratch_types=[
            pltpu.VMEM((_LANES,), jnp.int32),
            pltpu.VMEM((_LANES, 8, _D), jnp.float32),
            pltpu.VMEM((_LANES,), jnp.float32),
            pltpu.VMEM((_D,), jnp.float32),
            pltpu.SemaphoreType.DMA,
        ],
        compiler_params=pltpu.CompilerParams(use_tc_tiling_on_sc=True),
    )(functools.partial(_sc_bow_body, n_chunks=n_chunks))


def _tc_head_body(parts_ref, w_ref, b_ref, out_ref, *, scale):
    bow = jnp.sum(parts_ref[...], axis=0, keepdims=True) * scale
    logits = lax.dot_general(
        bow, w_ref[...], (((1,), (1,)), ((), ())),
        preferred_element_type=jnp.float32) + b_ref[...]
    m = jnp.max(logits, axis=-1, keepdims=True)
    s = logits - m
    lse = jnp.log(jnp.sum(jnp.exp(s), axis=-1, keepdims=True))
    out_ref[...] = s - lse


def kernel(input, emb_tensor, freq, W, b):
    L = input.shape[1]
    V = emb_tensor.shape[0]
    l_pad = ((L + _LANES - 1) // _LANES) * _LANES
    idx = jnp.concatenate(
        [input[0], jnp.zeros((l_pad - L,), jnp.int32)])
    emb3 = emb_tensor.reshape(V // 8, 8, _D)
    n_chunks = l_pad // _LANES
    parts = _make_sc_bow(n_chunks)(idx, emb3, freq)

    scale = 1.0 / (float(L) * 100000.0)
    out = pl.pallas_call(
        functools.partial(_tc_head_body, scale=scale),
        out_shape=jax.ShapeDtypeStruct((1, 2), jnp.float32),
    )(parts.reshape(n_chunks, _D), W, b.reshape(1, 2))
    return out
